# baseline (device time: 15142 ns/iter reference)
import jax
import jax.numpy as jnp
from jax import lax
from jax.experimental import pallas as pl
from jax.experimental.pallas import tpu as pltpu

N_DEV = 32
N_HALF = 2

_sem_signal = getattr(pl, "semaphore_signal", None) or pltpu.semaphore_signal
_sem_wait = getattr(pl, "semaphore_wait", None) or pltpu.semaphore_wait
_CompilerParams = getattr(pltpu, "CompilerParams", None) or pltpu.TPUCompilerParams
_DeviceIdType = getattr(pl, "DeviceIdType", None) or pltpu.DeviceIdType

_SEND_ORDER = sorted(range(1, N_DEV), key=lambda d: -min(d, N_DEV - d))
_RECV_ORDER = sorted(range(1, N_DEV), key=lambda d: min(d, N_DEV - d))


def kernel(x):
    m_per, n = x.shape
    ncol = n // N_HALF
    assert ncol * N_HALF == n and ncol % 128 == 0

    def body(
        x_hbm,
        out_hbm,
        xv_ref,
        comm_ref,
        res_ref,
        send_sems,
        recv_sems,
        load_sems,
        out_sem,
    ):
        my_pos = lax.axis_index("i")

        loads = [
            pltpu.make_async_copy(
                x_hbm.at[:, pl.ds(h * ncol, ncol)],
                xv_ref.at[h],
                load_sems.at[h],
            )
            for h in range(N_HALF)
        ]
        for ld in loads:
            ld.start()

        barrier_sem = pltpu.get_barrier_semaphore()
        _sem_signal(barrier_sem, inc=1)
        _sem_wait(barrier_sem, 1)

        sends = []
        for h in range(N_HALF):
            loads[h].wait()
            comm_ref[pl.ds(my_pos, 1), pl.ds(h * ncol, ncol)] = jnp.max(
                xv_ref[h], axis=0, keepdims=True
            )
            for d in _SEND_ORDER:
                s = pltpu.make_async_remote_copy(
                    src_ref=comm_ref.at[my_pos, pl.ds(h * ncol, ncol)],
                    dst_ref=comm_ref.at[my_pos, pl.ds(h * ncol, ncol)],
                    send_sem=send_sems.at[d, h],
                    recv_sem=recv_sems.at[my_pos, h],
                    device_id=((my_pos + d) % N_DEV,),
                    device_id_type=_DeviceIdType.MESH,
                )
                s.start()
                sends.append(s)

        for d in _RECV_ORDER:
            src_pos = (my_pos + d) % N_DEV
            for h in range(N_HALF):
                recv = pltpu.make_async_remote_copy(
                    src_ref=comm_ref.at[src_pos, pl.ds(h * ncol, ncol)],
                    dst_ref=comm_ref.at[src_pos, pl.ds(h * ncol, ncol)],
                    send_sem=send_sems.at[d, h],
                    recv_sem=recv_sems.at[src_pos, h],
                    device_id=(my_pos,),
                    device_id_type=_DeviceIdType.MESH,
                )
                recv.wait_recv()

        res_ref[:, :] = jnp.max(comm_ref[:, :], axis=0, keepdims=True)
        out_cp = pltpu.make_async_copy(res_ref, out_hbm, out_sem)
        out_cp.start()
        out_cp.wait()

        for s in sends:
            s.wait_send()

    return pl.pallas_call(
        body,
        out_shape=jax.ShapeDtypeStruct((1, n), x.dtype),
        in_specs=[pl.BlockSpec(memory_space=pltpu.MemorySpace.HBM)],
        out_specs=pl.BlockSpec(memory_space=pltpu.MemorySpace.HBM),
        scratch_shapes=[
            pltpu.VMEM((N_HALF, m_per, n // N_HALF), x.dtype),
            pltpu.VMEM((N_DEV, n), x.dtype),
            pltpu.VMEM((1, n), x.dtype),
            pltpu.SemaphoreType.DMA((N_DEV, N_HALF)),
            pltpu.SemaphoreType.DMA((N_DEV, N_HALF)),
            pltpu.SemaphoreType.DMA((N_HALF,)),
            pltpu.SemaphoreType.DMA,
        ],
        compiler_params=_CompilerParams(collective_id=0),
    )(x)


# device time: 13948 ns/iter; 1.0856x vs baseline; 1.0856x over previous
import jax
import jax.numpy as jnp
from jax import lax
from jax.experimental import pallas as pl
from jax.experimental.pallas import tpu as pltpu

N_DEV = 32

_sem_signal = getattr(pl, "semaphore_signal", None) or pltpu.semaphore_signal
_sem_wait = getattr(pl, "semaphore_wait", None) or pltpu.semaphore_wait
_CompilerParams = getattr(pltpu, "CompilerParams", None) or pltpu.TPUCompilerParams
_DeviceIdType = getattr(pl, "DeviceIdType", None) or pltpu.DeviceIdType

_SEND_ORDER = sorted(range(1, N_DEV), key=lambda d: -min(d, N_DEV - d))


def kernel(x):
    m_per, n = x.shape

    def body(x_ref, out_ref, comm_ref, send_sem, recv_sem):
        my_pos = lax.axis_index("i")

        barrier_sem = pltpu.get_barrier_semaphore()
        for d in range(1, N_DEV):
            _sem_signal(
                barrier_sem,
                inc=1,
                device_id=((my_pos + d) % N_DEV,),
                device_id_type=_DeviceIdType.MESH,
            )

        comm_ref[pl.ds(my_pos, 1), :] = jnp.max(
            x_ref[:, :], axis=0, keepdims=True
        )

        _sem_wait(barrier_sem, N_DEV - 1)

        for d in _SEND_ORDER:
            pltpu.make_async_remote_copy(
                src_ref=comm_ref.at[my_pos],
                dst_ref=comm_ref.at[my_pos],
                send_sem=send_sem,
                recv_sem=recv_sem,
                device_id=((my_pos + d) % N_DEV,),
                device_id_type=_DeviceIdType.MESH,
            ).start()

        wait_desc = pltpu.make_async_remote_copy(
            src_ref=comm_ref.at[my_pos],
            dst_ref=comm_ref.at[my_pos],
            send_sem=send_sem,
            recv_sem=recv_sem,
            device_id=(my_pos,),
            device_id_type=_DeviceIdType.MESH,
        )
        for _ in range(N_DEV - 1):
            wait_desc.wait_recv()

        out_ref[:, :] = jnp.max(comm_ref[:, :], axis=0, keepdims=True)

        for _ in range(N_DEV - 1):
            wait_desc.wait_send()

    return pl.pallas_call(
        body,
        out_shape=jax.ShapeDtypeStruct((1, n), x.dtype),
        in_specs=[pl.BlockSpec(memory_space=pltpu.MemorySpace.VMEM)],
        out_specs=pl.BlockSpec(memory_space=pltpu.MemorySpace.VMEM),
        scratch_shapes=[
            pltpu.VMEM((N_DEV, n), x.dtype),
            pltpu.SemaphoreType.DMA,
            pltpu.SemaphoreType.DMA,
        ],
        compiler_params=_CompilerParams(collective_id=0),
    )(x)
